# baseline (speedup 1.0000x reference)
import jax
import jax.numpy as jnp
from jax import lax
from jax.experimental import pallas as pl
from jax.experimental.pallas import tpu as pltpu
from jax.experimental.pallas import tpu_sc as plsc

N = 10000
E = 320000
D = 128

NC = 2
NS = 16
NW = NC * NS
C = 64
TCH = 160
EPW = TCH * C
EPAD = NW * EPW + 2 * C
NACC = 10016

RPT = 624
REM_BASE = NS * RPT
REM = N - REM_BASE

_mesh = plsc.VectorSubcoreMesh(
    core_axis_name="c", subcore_axis_name="s", num_cores=NC, num_subcores=NS
)


def _tile_rows_copy(s, fn):
    fn(s * RPT, RPT)

    @pl.when(s == NS - 1)
    def _():
        fn(REM_BASE, REM)


def _deg_body(dst_hbm, ones_hbm, zeros_hbm, onesC_hbm, out_hbm,
              id0, id1, id2, id3, ones_v, acc,
              sid0, sid1, sid2, sid3, sem):
    c = lax.axis_index("c")
    s = lax.axis_index("s")
    wid = s * NC + c

    @pl.when(c == 0)
    def _():
        _tile_rows_copy(s, lambda rb, nr: pltpu.sync_copy(
            ones_hbm.at[pl.ds(rb, nr)], acc.at[pl.ds(rb, nr)]))

    @pl.when(c != 0)
    def _():
        _tile_rows_copy(s, lambda rb, nr: pltpu.sync_copy(
            zeros_hbm.at[pl.ds(rb, nr)], acc.at[pl.ds(rb, nr)]))

    pltpu.sync_copy(onesC_hbm, ones_v)
    plsc.subcore_barrier()
    ebase = wid * EPW
    idx_d = (id0, id1, id2, id3)
    semid = (sid0, sid1, sid2, sid3)

    def idx_start(chunk, b):
        pltpu.async_copy(
            dst_hbm.at[pl.ds(ebase + chunk * C, C)], idx_d[b], semid[b])

    def idx_wait(chunk, b):
        pltpu.make_async_copy(
            dst_hbm.at[pl.ds(ebase + chunk * C, C)], idx_d[b], semid[b]).wait()

    def s_start(b):
        pltpu.async_copy(ones_v, acc.at[idx_d[b]], sem, add=True)

    def s_wait(b):
        pltpu.make_async_copy(ones_v, acc.at[idx_d[b]], sem).wait()

    idx_start(0, 0)
    idx_start(1, 1)
    idx_wait(0, 0)
    s_start(0)
    idx_start(2, 2)
    idx_wait(1, 1)
    s_start(1)
    idx_start(3, 3)

    @pl.loop(2, TCH - 2, step=4)
    def _(k):
        for j in range(4):
            b = (2 + j) % 4
            bw = j % 4
            idx_wait(k + j, b)
            s_start(b)
            s_wait(bw)
            idx_start(k + j + 2, bw)

    idx_wait(TCH - 2, 2)
    s_start(2)
    s_wait(0)
    idx_start(TCH, 0)
    idx_wait(TCH - 1, 3)
    s_start(3)
    s_wait(1)
    idx_start(TCH + 1, 1)
    s_wait(2)
    s_wait(3)
    idx_wait(TCH, 0)
    idx_wait(TCH + 1, 1)

    plsc.subcore_barrier()
    _tile_rows_copy(s, lambda rb, nr: pltpu.async_copy(
        acc.at[pl.ds(rb, nr)], out_hbm.at[c, pl.ds(rb, nr)], sem).wait())


@jax.jit
def _sc_degrees(dst, ones128, zeros128, onesC):
    kern = pl.kernel(
        _deg_body,
        out_type=jax.ShapeDtypeStruct((NC, N, D), jnp.float32),
        mesh=_mesh,
        scratch_types=(
            [pltpu.VMEM((C,), jnp.int32)] * 4
            + [pltpu.VMEM((C, D), jnp.float32),
               pltpu.VMEM_SHARED((NACC, D), jnp.float32)]
            + [pltpu.SemaphoreType.DMA] * 5
        ),
    )
    return kern(dst, ones128, zeros128, onesC)


def _agg_body(g_hbm, src_hbm, dst_hbm, zeros_hbm, out_hbm,
              is0, is1, is2, is3, id0, id1, id2, id3,
              r0, r1, r2, r3, acc,
              sis0, sis1, sis2, sis3, sid0, sid1, sid2, sid3,
              sg0, sg1, sg2, sg3, ssc0, ssc1, ssc2, ssc3, sem):
    c = lax.axis_index("c")
    s = lax.axis_index("s")
    wid = s * NC + c

    @pl.when(c == 0)
    def _():
        _tile_rows_copy(s, lambda rb, nr: pltpu.sync_copy(
            g_hbm.at[pl.ds(rb, nr)], acc.at[pl.ds(rb, nr)]))

    @pl.when(c != 0)
    def _():
        _tile_rows_copy(s, lambda rb, nr: pltpu.sync_copy(
            zeros_hbm.at[pl.ds(rb, nr)], acc.at[pl.ds(rb, nr)]))

    plsc.subcore_barrier()
    ebase = wid * EPW
    idx_s = (is0, is1, is2, is3)
    idx_d = (id0, id1, id2, id3)
    rows = (r0, r1, r2, r3)
    semis = (sis0, sis1, sis2, sis3)
    semid = (sid0, sid1, sid2, sid3)
    semg = (sg0, sg1, sg2, sg3)
    semsc = (ssc0, ssc1, ssc2, ssc3)

    def idx_start(chunk, b):
        base = ebase + chunk * C
        pltpu.async_copy(src_hbm.at[pl.ds(base, C)], idx_s[b], semis[b])
        pltpu.async_copy(dst_hbm.at[pl.ds(base, C)], idx_d[b], semid[b])

    def idx_wait(chunk, b):
        base = ebase + chunk * C
        pltpu.make_async_copy(src_hbm.at[pl.ds(base, C)], idx_s[b], semis[b]).wait()
        pltpu.make_async_copy(dst_hbm.at[pl.ds(base, C)], idx_d[b], semid[b]).wait()

    def g_start(b):
        pltpu.async_copy(g_hbm.at[idx_s[b]], rows[b], semg[b])

    def g_wait(b):
        pltpu.make_async_copy(g_hbm.at[idx_s[b]], rows[b], semg[b]).wait()

    def s_start(b):
        pltpu.async_copy(rows[b], acc.at[idx_d[b]], semsc[b], add=True)

    def s_wait(b):
        pltpu.make_async_copy(rows[b], acc.at[idx_d[b]], semsc[b]).wait()

    idx_start(0, 0)
    idx_start(1, 1)
    idx_wait(0, 0)
    g_start(0)
    g_wait(0)
    idx_wait(1, 1)
    g_start(1)
    s_start(0)
    idx_start(2, 2)
    g_wait(1)
    idx_wait(2, 2)
    g_start(2)
    s_start(1)
    idx_start(3, 3)

    @pl.loop(3, TCH - 1, step=4)
    def _(k):
        for j in range(4):
            b = (3 + j) % 4
            bm1 = (2 + j) % 4
            bw = j % 4
            g_wait(bm1)
            idx_wait(k + j, b)
            g_start(b)
            s_start(bm1)
            s_wait(bw)
            idx_start(k + j + 1, bw)

    g_wait(2)
    idx_wait(TCH - 1, 3)
    g_start(3)
    s_start(2)
    s_wait(0)
    idx_start(TCH, 0)
    g_wait(3)
    s_start(3)
    s_wait(1)
    s_wait(2)
    s_wait(3)
    idx_wait(TCH, 0)

    plsc.subcore_barrier()
    _tile_rows_copy(s, lambda rb, nr: pltpu.async_copy(
        acc.at[pl.ds(rb, nr)], out_hbm.at[c, pl.ds(rb, nr)], sem).wait())


@jax.jit
def _sc_aggregate(g, src, dst, zeros128):
    kern = pl.kernel(
        _agg_body,
        out_type=jax.ShapeDtypeStruct((NC, N, D), jnp.float32),
        mesh=_mesh,
        scratch_types=(
            [pltpu.VMEM((C,), jnp.int32)] * 8
            + [pltpu.VMEM((C, D), jnp.float32)] * 4
            + [pltpu.VMEM_SHARED((NACC, D), jnp.float32)]
            + [pltpu.SemaphoreType.DMA] * 17
        ),
    )
    return kern(g, src, dst, zeros128)


BM = 1000
GRID = N // BM


def _mm_body(x_ref, w_ref, o_ref):
    o_ref[...] = jnp.dot(x_ref[...], w_ref[...],
                         preferred_element_type=jnp.float32)


def _scale_body(u_ref, degp_ref, o_ref):
    deg = degp_ref[0, :, 0:1] + degp_ref[1, :, 0:1]
    o_ref[...] = u_ref[...] * lax.rsqrt(deg)


def _comb2_body(sp_ref, degp_ref, b_ref, w_ref, o_ref):
    deg = degp_ref[0, :, 0:1] + degp_ref[1, :, 0:1]
    dis = lax.rsqrt(deg)
    h = jnp.maximum((sp_ref[0] + sp_ref[1]) * dis + b_ref[...], 0.0)
    o_ref[...] = jnp.dot(h, w_ref[...],
                         preferred_element_type=jnp.float32) * dis


def _out_body(sp_ref, degp_ref, b_ref, w_ref, bl_ref, o_ref):
    deg = degp_ref[0, :, 0:1] + degp_ref[1, :, 0:1]
    dis = lax.rsqrt(deg)
    h = (sp_ref[0] + sp_ref[1]) * dis + b_ref[...]
    o_ref[...] = jnp.dot(h, w_ref[...],
                         preferred_element_type=jnp.float32) + bl_ref[...]


_spec_rows = pl.BlockSpec((BM, D), lambda i: (i, 0))
_spec_part = pl.BlockSpec((NC, BM, D), lambda i: (0, i, 0))
_spec_w = pl.BlockSpec((D, D), lambda i: (0, 0))
_spec_b = pl.BlockSpec((1, D), lambda i: (0, 0))
_f32 = jnp.float32


@jax.jit
def _tc_mm(x, w):
    return pl.pallas_call(
        _mm_body, grid=(GRID,),
        in_specs=[_spec_rows, _spec_w], out_specs=_spec_rows,
        out_shape=jax.ShapeDtypeStruct((N, D), _f32),
    )(x, w)


@jax.jit
def _tc_scale(u, degp):
    return pl.pallas_call(
        _scale_body, grid=(GRID,),
        in_specs=[_spec_rows, _spec_part], out_specs=_spec_rows,
        out_shape=jax.ShapeDtypeStruct((N, D), _f32),
    )(u, degp)


@jax.jit
def _tc_comb2(sp, degp, b1, w2):
    return pl.pallas_call(
        _comb2_body, grid=(GRID,),
        in_specs=[_spec_part, _spec_part, _spec_b, _spec_w],
        out_specs=_spec_rows,
        out_shape=jax.ShapeDtypeStruct((N, D), _f32),
    )(sp, degp, b1, w2)


@jax.jit
def _tc_out(sp, degp, b2, wl, bl):
    return pl.pallas_call(
        _out_body, grid=(GRID,),
        in_specs=[_spec_part, _spec_part, _spec_b, _spec_w, _spec_b],
        out_specs=_spec_rows,
        out_shape=jax.ShapeDtypeStruct((N, D), _f32),
    )(sp, degp, b2, wl, bl)


def kernel(x, edge_index, W1, b1, W2, b2, Wl, bl):
    npad = EPAD - E
    src = jnp.concatenate([edge_index[0],
                           jnp.zeros((npad,), edge_index.dtype)])
    dst = jnp.concatenate([edge_index[1],
                           N + (jnp.arange(npad, dtype=edge_index.dtype) % 8)])
    zeros128 = jnp.zeros((N, D), _f32)
    ones128 = jnp.ones((N, D), _f32)
    onesC = jnp.ones((C, D), _f32)
    b1r = b1.reshape(1, D)
    b2r = b2.reshape(1, D)
    blr = bl.reshape(1, D)

    degp = _sc_degrees(dst, ones128, zeros128, onesC)
    u1 = _tc_mm(x, W1)
    g1 = _tc_scale(u1, degp)
    s1 = _sc_aggregate(g1, src, dst, zeros128)
    g2 = _tc_comb2(s1, degp, b1r, W2)
    s2 = _sc_aggregate(g2, src, dst, zeros128)
    return _tc_out(s2, degp, b2r, Wl, blr)


# speedup vs baseline: 2.6415x; 2.6415x over previous
import jax
import jax.numpy as jnp
from jax import lax
from jax.experimental import pallas as pl
from jax.experimental.pallas import tpu as pltpu
from jax.experimental.pallas import tpu_sc as plsc

N = 10000
E = 320000
D = 128

NC = 2
NS = 16
NW = NC * NS
C = 80
TCH = 125
EPW = TCH * C
EPAD = NW * EPW + 2 * C
NACC = 10016

RPT = 624
REM_BASE = NS * RPT
REM = N - REM_BASE

_mesh = plsc.VectorSubcoreMesh(
    core_axis_name="c", subcore_axis_name="s", num_cores=NC, num_subcores=NS
)


def _tile_rows_copy(s, fn):
    fn(s * RPT, RPT)

    @pl.when(s == NS - 1)
    def _():
        fn(REM_BASE, REM)


def _deg_body(dst_hbm, ones_hbm, zeros_hbm, onesC_hbm, out_hbm,
              id0, id1, id2, id3, ones_v, acc,
              sid0, sid1, sid2, sid3, sem):
    c = lax.axis_index("c")
    s = lax.axis_index("s")
    wid = s * NC + c

    @pl.when(c == 0)
    def _():
        _tile_rows_copy(s, lambda rb, nr: pltpu.sync_copy(
            ones_hbm.at[pl.ds(rb, nr)], acc.at[pl.ds(rb, nr)]))

    @pl.when(c != 0)
    def _():
        _tile_rows_copy(s, lambda rb, nr: pltpu.sync_copy(
            zeros_hbm.at[pl.ds(rb, nr)], acc.at[pl.ds(rb, nr)]))

    pltpu.sync_copy(onesC_hbm, ones_v)
    plsc.subcore_barrier()
    ebase = wid * EPW
    idx_d = (id0, id1, id2, id3)
    semid = (sid0, sid1, sid2, sid3)

    def idx_start(chunk, b):
        pltpu.async_copy(
            dst_hbm.at[pl.ds(ebase + chunk * C, C)], idx_d[b], semid[b])

    def idx_wait(chunk, b):
        pltpu.make_async_copy(
            dst_hbm.at[pl.ds(ebase + chunk * C, C)], idx_d[b], semid[b]).wait()

    def s_start(b):
        pltpu.async_copy(ones_v, acc.at[idx_d[b]], sem, add=True)

    def s_wait(b):
        pltpu.make_async_copy(ones_v, acc.at[idx_d[b]], sem).wait()

    idx_start(0, 0)
    idx_start(1, 1)
    idx_wait(0, 0)
    s_start(0)
    idx_start(2, 2)
    idx_wait(1, 1)
    s_start(1)
    idx_start(3, 3)

    @pl.loop(2, TCH - 3, step=4)
    def _(k):
        for j in range(4):
            b = (2 + j) % 4
            bw = j % 4
            idx_wait(k + j, b)
            s_start(b)
            s_wait(bw)
            idx_start(k + j + 2, bw)

    idx_wait(TCH - 3, 2)
    s_start(2)
    s_wait(0)
    idx_start(TCH - 1, 0)
    idx_wait(TCH - 2, 3)
    s_start(3)
    s_wait(1)
    idx_start(TCH, 1)
    idx_wait(TCH - 1, 0)
    s_start(0)
    s_wait(2)
    idx_start(TCH + 1, 2)
    s_wait(3)
    s_wait(0)
    idx_wait(TCH, 1)
    idx_wait(TCH + 1, 2)

    plsc.subcore_barrier()
    _tile_rows_copy(s, lambda rb, nr: pltpu.async_copy(
        acc.at[pl.ds(rb, nr)], out_hbm.at[c, pl.ds(rb, nr)], sem).wait())


@jax.jit
def _sc_degrees(dst, ones128, zeros128, onesC):
    kern = pl.kernel(
        _deg_body,
        out_type=jax.ShapeDtypeStruct((NC, N, D), jnp.float32),
        mesh=_mesh,
        scratch_types=(
            [pltpu.VMEM((C,), jnp.int32)] * 4
            + [pltpu.VMEM((C, D), jnp.float32),
               pltpu.VMEM_SHARED((NACC, D), jnp.float32)]
            + [pltpu.SemaphoreType.DMA] * 5
        ),
    )
    return kern(dst, ones128, zeros128, onesC)


def _agg_body(g_hbm, src_hbm, dst_hbm, zeros_hbm, out_hbm,
              is0, is1, is2, is3, id0, id1, id2, id3,
              r0, r1, r2, r3, acc,
              sis0, sis1, sis2, sis3, sid0, sid1, sid2, sid3,
              sg0, sg1, sg2, sg3, ssc0, ssc1, ssc2, ssc3, sem):
    c = lax.axis_index("c")
    s = lax.axis_index("s")
    wid = s * NC + c

    @pl.when(c == 0)
    def _():
        _tile_rows_copy(s, lambda rb, nr: pltpu.sync_copy(
            g_hbm.at[pl.ds(rb, nr)], acc.at[pl.ds(rb, nr)]))

    @pl.when(c != 0)
    def _():
        _tile_rows_copy(s, lambda rb, nr: pltpu.sync_copy(
            zeros_hbm.at[pl.ds(rb, nr)], acc.at[pl.ds(rb, nr)]))

    plsc.subcore_barrier()
    ebase = wid * EPW
    idx_s = (is0, is1, is2, is3)
    idx_d = (id0, id1, id2, id3)
    rows = (r0, r1, r2, r3)
    semis = (sis0, sis1, sis2, sis3)
    semid = (sid0, sid1, sid2, sid3)
    semg = (sg0, sg1, sg2, sg3)
    semsc = (ssc0, ssc1, ssc2, ssc3)

    def idx_start(chunk, b):
        base = ebase + chunk * C
        pltpu.async_copy(src_hbm.at[pl.ds(base, C)], idx_s[b], semis[b])
        pltpu.async_copy(dst_hbm.at[pl.ds(base, C)], idx_d[b], semid[b])

    def idx_wait(chunk, b):
        base = ebase + chunk * C
        pltpu.make_async_copy(src_hbm.at[pl.ds(base, C)], idx_s[b], semis[b]).wait()
        pltpu.make_async_copy(dst_hbm.at[pl.ds(base, C)], idx_d[b], semid[b]).wait()

    def g_start(b):
        pltpu.async_copy(g_hbm.at[idx_s[b]], rows[b], semg[b])

    def g_wait(b):
        pltpu.make_async_copy(g_hbm.at[idx_s[b]], rows[b], semg[b]).wait()

    def s_start(b):
        pltpu.async_copy(rows[b], acc.at[idx_d[b]], semsc[b], add=True)

    def s_wait(b):
        pltpu.make_async_copy(rows[b], acc.at[idx_d[b]], semsc[b]).wait()

    idx_start(0, 0)
    idx_start(1, 1)
    idx_wait(0, 0)
    g_start(0)
    g_wait(0)
    idx_wait(1, 1)
    g_start(1)
    s_start(0)
    idx_start(2, 2)
    g_wait(1)
    idx_wait(2, 2)
    g_start(2)
    s_start(1)
    idx_start(3, 3)

    @pl.loop(3, TCH - 2, step=4)
    def _(k):
        for j in range(4):
            b = (3 + j) % 4
            bm1 = (2 + j) % 4
            bw = j % 4
            g_wait(bm1)
            idx_wait(k + j, b)
            g_start(b)
            s_start(bm1)
            s_wait(bw)
            idx_start(k + j + 1, bw)

    g_wait(2)
    idx_wait(TCH - 2, 3)
    g_start(3)
    s_start(2)
    s_wait(0)
    idx_start(TCH - 1, 0)
    g_wait(3)
    idx_wait(TCH - 1, 0)
    g_start(0)
    s_start(3)
    s_wait(1)
    idx_start(TCH, 1)
    g_wait(0)
    s_start(0)
    s_wait(2)
    s_wait(3)
    s_wait(0)
    idx_wait(TCH, 1)

    plsc.subcore_barrier()
    _tile_rows_copy(s, lambda rb, nr: pltpu.async_copy(
        acc.at[pl.ds(rb, nr)], out_hbm.at[c, pl.ds(rb, nr)], sem).wait())


@jax.jit
def _sc_aggregate(g, src, dst, zeros128):
    kern = pl.kernel(
        _agg_body,
        out_type=jax.ShapeDtypeStruct((NC, N, D), jnp.float32),
        mesh=_mesh,
        scratch_types=(
            [pltpu.VMEM((C,), jnp.int32)] * 8
            + [pltpu.VMEM((C, D), jnp.float32)] * 4
            + [pltpu.VMEM_SHARED((NACC, D), jnp.float32)]
            + [pltpu.SemaphoreType.DMA] * 17
        ),
    )
    return kern(g, src, dst, zeros128)


BM = 1000
GRID = N // BM


def _mm_body(x_ref, w_ref, o_ref):
    o_ref[...] = jnp.dot(x_ref[...], w_ref[...],
                         preferred_element_type=jnp.float32)


def _scale_body(u_ref, degp_ref, o_ref):
    deg = degp_ref[0, :, 0:1] + degp_ref[1, :, 0:1]
    o_ref[...] = u_ref[...] * lax.rsqrt(deg)


def _comb2_body(sp_ref, degp_ref, b_ref, w_ref, o_ref):
    deg = degp_ref[0, :, 0:1] + degp_ref[1, :, 0:1]
    dis = lax.rsqrt(deg)
    h = jnp.maximum((sp_ref[0] + sp_ref[1]) * dis + b_ref[...], 0.0)
    o_ref[...] = jnp.dot(h, w_ref[...],
                         preferred_element_type=jnp.float32) * dis


def _out_body(sp_ref, degp_ref, b_ref, w_ref, bl_ref, o_ref):
    deg = degp_ref[0, :, 0:1] + degp_ref[1, :, 0:1]
    dis = lax.rsqrt(deg)
    h = (sp_ref[0] + sp_ref[1]) * dis + b_ref[...]
    o_ref[...] = jnp.dot(h, w_ref[...],
                         preferred_element_type=jnp.float32) + bl_ref[...]


_spec_rows = pl.BlockSpec((BM, D), lambda i: (i, 0))
_spec_part = pl.BlockSpec((NC, BM, D), lambda i: (0, i, 0))
_spec_w = pl.BlockSpec((D, D), lambda i: (0, 0))
_spec_b = pl.BlockSpec((1, D), lambda i: (0, 0))
_f32 = jnp.float32


@jax.jit
def _tc_mm(x, w):
    return pl.pallas_call(
        _mm_body, grid=(GRID,),
        in_specs=[_spec_rows, _spec_w], out_specs=_spec_rows,
        out_shape=jax.ShapeDtypeStruct((N, D), _f32),
    )(x, w)


@jax.jit
def _tc_scale(u, degp):
    return pl.pallas_call(
        _scale_body, grid=(GRID,),
        in_specs=[_spec_rows, _spec_part], out_specs=_spec_rows,
        out_shape=jax.ShapeDtypeStruct((N, D), _f32),
    )(u, degp)


@jax.jit
def _tc_comb2(sp, degp, b1, w2):
    return pl.pallas_call(
        _comb2_body, grid=(GRID,),
        in_specs=[_spec_part, _spec_part, _spec_b, _spec_w],
        out_specs=_spec_rows,
        out_shape=jax.ShapeDtypeStruct((N, D), _f32),
    )(sp, degp, b1, w2)


@jax.jit
def _tc_out(sp, degp, b2, wl, bl):
    return pl.pallas_call(
        _out_body, grid=(GRID,),
        in_specs=[_spec_part, _spec_part, _spec_b, _spec_w, _spec_b],
        out_specs=_spec_rows,
        out_shape=jax.ShapeDtypeStruct((N, D), _f32),
    )(sp, degp, b2, wl, bl)


def kernel(x, edge_index, W1, b1, W2, b2, Wl, bl):
    npad = EPAD - E
    src = jnp.concatenate([edge_index[0],
                           jnp.zeros((npad,), edge_index.dtype)])
    dst = jnp.concatenate([edge_index[1],
                           N + (jnp.arange(npad, dtype=edge_index.dtype) % 8)])
    zeros128 = jnp.zeros((N, D), _f32)
    ones128 = jnp.ones((N, D), _f32)
    onesC = jnp.ones((C, D), _f32)
    b1r = b1.reshape(1, D)
    b2r = b2.reshape(1, D)
    blr = bl.reshape(1, D)

    degp = _sc_degrees(dst, ones128, zeros128, onesC)
    u1 = _tc_mm(x, W1)
    g1 = _tc_scale(u1, degp)
    s1 = _sc_aggregate(g1, src, dst, zeros128)
    g2 = _tc_comb2(s1, degp, b1r, W2)
    s2 = _sc_aggregate(g2, src, dst, zeros128)
    return _tc_out(s2, degp, b2r, Wl, blr)


# speedup vs baseline: 3.0836x; 1.1674x over previous
import jax
import jax.numpy as jnp
from jax import lax
from jax.experimental import pallas as pl
from jax.experimental.pallas import tpu as pltpu
from jax.experimental.pallas import tpu_sc as plsc

N = 10000
E = 320000
D = 128

NC = 2
NS = 16
NW = NC * NS
C = 80
TCH = 125
EPW = TCH * C
EPAD = NW * EPW + 2 * C
NACC = 10016

RPT = 624
REM_BASE = NS * RPT
REM = N - REM_BASE

_mesh = plsc.VectorSubcoreMesh(
    core_axis_name="c", subcore_axis_name="s", num_cores=NC, num_subcores=NS
)


def _tile_rows_copy(s, fn):
    fn(s * RPT, RPT)

    @pl.when(s == NS - 1)
    def _():
        fn(REM_BASE, REM)


def _deg_body(dst_hbm, ones_hbm, zeros_hbm, onesC_hbm, out_hbm,
              id0, id1, id2, id3, ones_v, acc,
              sid0, sid1, sid2, sid3, sem):
    c = lax.axis_index("c")
    s = lax.axis_index("s")
    wid = s * NC + c

    @pl.when(c == 0)
    def _():
        _tile_rows_copy(s, lambda rb, nr: pltpu.sync_copy(
            ones_hbm.at[pl.ds(rb, nr)], acc.at[pl.ds(rb, nr)]))

    @pl.when(c != 0)
    def _():
        _tile_rows_copy(s, lambda rb, nr: pltpu.sync_copy(
            zeros_hbm.at[pl.ds(rb, nr)], acc.at[pl.ds(rb, nr)]))

    pltpu.sync_copy(onesC_hbm, ones_v)
    plsc.subcore_barrier()
    ebase = wid * EPW
    idx_d = (id0, id1, id2, id3)
    semid = (sid0, sid1, sid2, sid3)

    def idx_start(chunk, b):
        pltpu.async_copy(
            dst_hbm.at[pl.ds(ebase + chunk * C, C)], idx_d[b], semid[b])

    def idx_wait(chunk, b):
        pltpu.make_async_copy(
            dst_hbm.at[pl.ds(ebase + chunk * C, C)], idx_d[b], semid[b]).wait()

    def s_start(b):
        pltpu.async_copy(ones_v, acc.at[idx_d[b]], sem, add=True)

    def s_wait(b):
        pltpu.make_async_copy(ones_v, acc.at[idx_d[b]], sem).wait()

    idx_start(0, 0)
    idx_start(1, 1)
    idx_wait(0, 0)
    s_start(0)
    idx_start(2, 2)
    idx_wait(1, 1)
    s_start(1)
    idx_start(3, 3)

    @pl.loop(2, TCH - 3, step=4)
    def _(k):
        for j in range(4):
            b = (2 + j) % 4
            bw = j % 4
            idx_wait(k + j, b)
            s_start(b)
            s_wait(bw)
            idx_start(k + j + 2, bw)

    idx_wait(TCH - 3, 2)
    s_start(2)
    s_wait(0)
    idx_start(TCH - 1, 0)
    idx_wait(TCH - 2, 3)
    s_start(3)
    s_wait(1)
    idx_start(TCH, 1)
    idx_wait(TCH - 1, 0)
    s_start(0)
    s_wait(2)
    idx_start(TCH + 1, 2)
    s_wait(3)
    s_wait(0)
    idx_wait(TCH, 1)
    idx_wait(TCH + 1, 2)

    plsc.subcore_barrier()
    _tile_rows_copy(s, lambda rb, nr: pltpu.async_copy(
        acc.at[pl.ds(rb, nr)], out_hbm.at[c, pl.ds(rb, nr)], sem).wait())


@jax.jit
def _sc_degrees(dst, ones128, zeros128, onesC):
    kern = pl.kernel(
        _deg_body,
        out_type=jax.ShapeDtypeStruct((NC, N, D), jnp.float32),
        mesh=_mesh,
        scratch_types=(
            [pltpu.VMEM((C,), jnp.int32)] * 4
            + [pltpu.VMEM((C, D), jnp.float32),
               pltpu.VMEM_SHARED((NACC, D), jnp.float32)]
            + [pltpu.SemaphoreType.DMA] * 5
        ),
    )
    return kern(dst, ones128, zeros128, onesC)


def _agg_body(g_hbm, src_hbm, dst_hbm, zeros_hbm, out_hbm,
              is0, is1, is2, is3, id0, id1, id2, id3,
              r0, r1, r2, r3, acc,
              sis0, sis1, sis2, sis3, sid0, sid1, sid2, sid3,
              sg0, sg1, sg2, sg3, ssc0, ssc1, ssc2, ssc3, sem):
    c = lax.axis_index("c")
    s = lax.axis_index("s")
    wid = s * NC + c

    @pl.when(c == 0)
    def _():
        _tile_rows_copy(s, lambda rb, nr: pltpu.sync_copy(
            g_hbm.at[pl.ds(rb, nr)], acc.at[pl.ds(rb, nr)]))

    @pl.when(c != 0)
    def _():
        _tile_rows_copy(s, lambda rb, nr: pltpu.sync_copy(
            zeros_hbm.at[pl.ds(rb, nr)], acc.at[pl.ds(rb, nr)]))

    plsc.subcore_barrier()
    ebase = wid * EPW
    idx_s = (is0, is1, is2, is3)
    idx_d = (id0, id1, id2, id3)
    rows = (r0, r1, r2, r3)
    semis = (sis0, sis1, sis2, sis3)
    semid = (sid0, sid1, sid2, sid3)
    semg = (sg0, sg1, sg2, sg3)
    semsc = (ssc0, ssc1, ssc2, ssc3)

    def idx_start(chunk, b):
        base = ebase + chunk * C
        pltpu.async_copy(src_hbm.at[pl.ds(base, C)], idx_s[b], semis[b])
        pltpu.async_copy(dst_hbm.at[pl.ds(base, C)], idx_d[b], semid[b])

    def idx_wait(chunk, b):
        base = ebase + chunk * C
        pltpu.make_async_copy(src_hbm.at[pl.ds(base, C)], idx_s[b], semis[b]).wait()
        pltpu.make_async_copy(dst_hbm.at[pl.ds(base, C)], idx_d[b], semid[b]).wait()

    def g_start(b):
        pltpu.async_copy(g_hbm.at[idx_s[b]], rows[b], semg[b])

    def g_wait(b):
        pltpu.make_async_copy(g_hbm.at[idx_s[b]], rows[b], semg[b]).wait()

    def s_start(b):
        pltpu.async_copy(rows[b], acc.at[idx_d[b]], semsc[b], add=True)

    def s_wait(b):
        pltpu.make_async_copy(rows[b], acc.at[idx_d[b]], semsc[b]).wait()

    idx_start(0, 0)
    idx_start(1, 1)
    idx_start(2, 2)
    idx_start(3, 3)
    idx_wait(0, 0)
    g_start(0)
    idx_wait(1, 1)
    g_start(1)
    g_wait(0)
    s_start(0)
    idx_wait(2, 2)
    g_start(2)
    g_wait(1)
    s_start(1)

    @pl.loop(2, TCH - 3, step=4)
    def _(k):
        for j in range(4):
            c4 = (2 + j) % 4
            idx_wait(k + j + 1, (c4 + 1) % 4)
            g_start((c4 + 1) % 4)
            g_wait(c4)
            s_start(c4)
            s_wait((c4 + 2) % 4)
            idx_start(k + j + 2, (c4 + 2) % 4)

    idx_wait(TCH - 2, 3)
    g_start(3)
    g_wait(2)
    s_start(2)
    s_wait(0)
    idx_start(TCH - 1, 0)
    idx_wait(TCH - 1, 0)
    g_start(0)
    g_wait(3)
    s_start(3)
    s_wait(1)
    idx_start(TCH, 1)
    g_wait(0)
    s_start(0)
    s_wait(2)
    s_wait(3)
    s_wait(0)
    idx_wait(TCH, 1)

    plsc.subcore_barrier()
    _tile_rows_copy(s, lambda rb, nr: pltpu.async_copy(
        acc.at[pl.ds(rb, nr)], out_hbm.at[c, pl.ds(rb, nr)], sem).wait())


@jax.jit
def _sc_aggregate(g, src, dst, zeros128):
    kern = pl.kernel(
        _agg_body,
        out_type=jax.ShapeDtypeStruct((NC, N, D), jnp.float32),
        mesh=_mesh,
        scratch_types=(
            [pltpu.VMEM((C,), jnp.int32)] * 8
            + [pltpu.VMEM((C, D), jnp.float32)] * 4
            + [pltpu.VMEM_SHARED((NACC, D), jnp.float32)]
            + [pltpu.SemaphoreType.DMA] * 17
        ),
    )
    return kern(g, src, dst, zeros128)


BM = 1000
GRID = N // BM


def _mm_body(x_ref, w_ref, o_ref):
    o_ref[...] = jnp.dot(x_ref[...], w_ref[...],
                         preferred_element_type=jnp.float32)


def _scale_body(u_ref, degp_ref, o_ref):
    deg = degp_ref[0, :, 0:1] + degp_ref[1, :, 0:1]
    o_ref[...] = u_ref[...] * lax.rsqrt(deg)


def _comb2_body(sp_ref, degp_ref, b_ref, w_ref, o_ref):
    deg = degp_ref[0, :, 0:1] + degp_ref[1, :, 0:1]
    dis = lax.rsqrt(deg)
    h = jnp.maximum((sp_ref[0] + sp_ref[1]) * dis + b_ref[...], 0.0)
    o_ref[...] = jnp.dot(h, w_ref[...],
                         preferred_element_type=jnp.float32) * dis


def _out_body(sp_ref, degp_ref, b_ref, w_ref, bl_ref, o_ref):
    deg = degp_ref[0, :, 0:1] + degp_ref[1, :, 0:1]
    dis = lax.rsqrt(deg)
    h = (sp_ref[0] + sp_ref[1]) * dis + b_ref[...]
    o_ref[...] = jnp.dot(h, w_ref[...],
                         preferred_element_type=jnp.float32) + bl_ref[...]


_spec_rows = pl.BlockSpec((BM, D), lambda i: (i, 0))
_spec_part = pl.BlockSpec((NC, BM, D), lambda i: (0, i, 0))
_spec_w = pl.BlockSpec((D, D), lambda i: (0, 0))
_spec_b = pl.BlockSpec((1, D), lambda i: (0, 0))
_f32 = jnp.float32


@jax.jit
def _tc_mm(x, w):
    return pl.pallas_call(
        _mm_body, grid=(GRID,),
        in_specs=[_spec_rows, _spec_w], out_specs=_spec_rows,
        out_shape=jax.ShapeDtypeStruct((N, D), _f32),
    )(x, w)


@jax.jit
def _tc_scale(u, degp):
    return pl.pallas_call(
        _scale_body, grid=(GRID,),
        in_specs=[_spec_rows, _spec_part], out_specs=_spec_rows,
        out_shape=jax.ShapeDtypeStruct((N, D), _f32),
    )(u, degp)


@jax.jit
def _tc_comb2(sp, degp, b1, w2):
    return pl.pallas_call(
        _comb2_body, grid=(GRID,),
        in_specs=[_spec_part, _spec_part, _spec_b, _spec_w],
        out_specs=_spec_rows,
        out_shape=jax.ShapeDtypeStruct((N, D), _f32),
    )(sp, degp, b1, w2)


@jax.jit
def _tc_out(sp, degp, b2, wl, bl):
    return pl.pallas_call(
        _out_body, grid=(GRID,),
        in_specs=[_spec_part, _spec_part, _spec_b, _spec_w, _spec_b],
        out_specs=_spec_rows,
        out_shape=jax.ShapeDtypeStruct((N, D), _f32),
    )(sp, degp, b2, wl, bl)


def kernel(x, edge_index, W1, b1, W2, b2, Wl, bl):
    npad = EPAD - E
    src = jnp.concatenate([edge_index[0],
                           jnp.zeros((npad,), edge_index.dtype)])
    dst = jnp.concatenate([edge_index[1],
                           N + (jnp.arange(npad, dtype=edge_index.dtype) % 8)])
    zeros128 = jnp.zeros((N, D), _f32)
    ones128 = jnp.ones((N, D), _f32)
    onesC = jnp.ones((C, D), _f32)
    b1r = b1.reshape(1, D)
    b2r = b2.reshape(1, D)
    blr = bl.reshape(1, D)

    degp = _sc_degrees(dst, ones128, zeros128, onesC)
    u1 = _tc_mm(x, W1)
    g1 = _tc_scale(u1, degp)
    s1 = _sc_aggregate(g1, src, dst, zeros128)
    g2 = _tc_comb2(s1, degp, b1r, W2)
    s2 = _sc_aggregate(g2, src, dst, zeros128)
    return _tc_out(s2, degp, b2r, Wl, blr)
